# baseline (speedup 1.0000x reference)
import functools

import jax
import jax.numpy as jnp
from jax.experimental import pallas as pl
from jax.experimental.pallas import tpu as pltpu

_N = 10000
_H = 8
_HG = 4
_NGRP = _H // _HG
_C = 128
_D = 256
_G = 64
_NCLS = 16
_E_RAW = 160000
_E_TOT = _E_RAW + _N
_EBLK = 10000
_NSTEP = _E_TOT // _EBLK
_NEG = 0.2
_NODE_BLK = 1000


def _proj_kernel(x_ref, w_ref, h_ref):
    h = jnp.dot(x_ref[...], w_ref[...], preferred_element_type=jnp.float32)
    h_ref[...] = h.reshape(h.shape[0], _H, _C)


def _gat_kernel(src_ref, dst_ref, h_ref, attl_ref, attr_ref, batch_ref,
                bias_ref, linw_ref, linb_ref, prev_ref, out_ref,
                acc_ref, den_ref, *, last):
    step = pl.program_id(0)

    @pl.when(step == 0)
    def _init():
        acc_ref[...] = jnp.zeros_like(acc_ref)
        den_ref[...] = jnp.zeros_like(den_ref)

    attl = attl_ref[0, 0]
    attr = attr_ref[0, 0]
    base = step * _EBLK

    def body(i, carry):
        s = src_ref[0, 0, i]
        d = dst_ref[0, 0, i]
        hs = h_ref[0, s]
        hd = h_ref[0, d]
        dp = jnp.sum(hs * hd, axis=-1)
        al = jnp.sum(hs * attl, axis=-1)
        ar = jnp.sum(hd * attr, axis=-1)
        a = (al + ar) * jax.nn.sigmoid(dp)
        a = jnp.where(a > 0, a, _NEG * a)
        valid = jnp.logical_or(s != d, base + i >= _E_RAW)
        e = jnp.where(valid, jnp.exp(a), 0.0)
        acc_ref[d] = acc_ref[d] + hs * e[:, None]
        den_ref[pl.ds(d, 1), :] = den_ref[pl.ds(d, 1), :] + e[None, :]
        return carry

    jax.lax.fori_loop(0, _EBLK, body, 0)

    @pl.when(step == _NSTEP - 1)
    def _finish():
        batch = batch_ref[...]
        gids = jax.lax.broadcasted_iota(jnp.int32, (_G, _N), 0)
        onehot = (gids == batch).astype(jnp.float32)
        counts = jnp.clip(jnp.sum(onehot, axis=1, keepdims=True), 1.0, None)
        den = den_ref[...]
        res = jnp.zeros((_G, _NCLS), jnp.float32)
        for h in range(_HG):
            o = acc_ref[:, h, :] / (den[:, h][:, None] + 1e-16)
            o = o + bias_ref[0, 0, h]
            o = jnp.where(o > 0, o, jnp.exp(jnp.minimum(o, 0.0)) - 1.0)
            pooled = jnp.dot(onehot, o, preferred_element_type=jnp.float32)
            res = res + jnp.dot(pooled / counts, linw_ref[h],
                                preferred_element_type=jnp.float32)
        res = res + prev_ref[...]
        if last:
            res = res + linb_ref[...]
        out_ref[...] = res


@jax.jit
def kernel(x, edge_index, batch, W, att_l, att_r, conv_bias, lin_W, lin_b):
    loop = jnp.arange(_N, dtype=edge_index.dtype)
    src = jnp.concatenate([edge_index[0], loop]).reshape(_NSTEP, 1, _EBLK)
    dst = jnp.concatenate([edge_index[1], loop]).reshape(_NSTEP, 1, _EBLK)
    attl = att_l.reshape(_NGRP, 1, _HG, _C)
    attr = att_r.reshape(_NGRP, 1, _HG, _C)
    bias = conv_bias.reshape(_NGRP, 1, _HG, _C)
    linw = lin_W.reshape(_NGRP, _HG, _C, _NCLS)
    linb = lin_b.reshape(1, _NCLS)
    batch2 = batch.reshape(1, _N)

    h = pl.pallas_call(
        _proj_kernel,
        grid=(_N // _NODE_BLK,),
        in_specs=[
            pl.BlockSpec((_NODE_BLK, _D), lambda i: (i, 0)),
            pl.BlockSpec((_D, _H * _C), lambda i: (0, 0)),
        ],
        out_specs=pl.BlockSpec((_NODE_BLK, _H, _C), lambda i: (i, 0, 0)),
        out_shape=jax.ShapeDtypeStruct((_N, _H, _C), jnp.float32),
    )(x, W)
    h4 = jnp.transpose(h.reshape(_N, _NGRP, _HG, _C), (1, 0, 2, 3))

    out = jnp.zeros((_G, _NCLS), jnp.float32)
    for g in range(_NGRP):
        out = pl.pallas_call(
            functools.partial(_gat_kernel, last=(g == _NGRP - 1)),
            grid=(_NSTEP,),
            in_specs=[
                pl.BlockSpec((1, 1, _EBLK), lambda i: (i, 0, 0),
                             memory_space=pltpu.SMEM),
                pl.BlockSpec((1, 1, _EBLK), lambda i: (i, 0, 0),
                             memory_space=pltpu.SMEM),
                pl.BlockSpec((1, _N, _HG, _C), lambda i: (0, 0, 0, 0)),
                pl.BlockSpec((1, 1, _HG, _C), lambda i: (0, 0, 0, 0)),
                pl.BlockSpec((1, 1, _HG, _C), lambda i: (0, 0, 0, 0)),
                pl.BlockSpec((1, _N), lambda i: (0, 0)),
                pl.BlockSpec((1, 1, _HG, _C), lambda i: (0, 0, 0, 0)),
                pl.BlockSpec((_HG, _C, _NCLS), lambda i: (0, 0, 0)),
                pl.BlockSpec((1, _NCLS), lambda i: (0, 0)),
                pl.BlockSpec((_G, _NCLS), lambda i: (0, 0)),
            ],
            out_specs=pl.BlockSpec((_G, _NCLS), lambda i: (0, 0)),
            out_shape=jax.ShapeDtypeStruct((_G, _NCLS), jnp.float32),
            scratch_shapes=[
                pltpu.VMEM((_N, _HG, _C), jnp.float32),
                pltpu.VMEM((_N, _HG), jnp.float32),
            ],
        )(src, dst, h4[g:g + 1], attl[g:g + 1], attr[g:g + 1], batch2,
          bias[g:g + 1], linw[g], linb, out)
    return out
